# baseline (speedup 1.0000x reference)
import functools

import jax
import jax.numpy as jnp
from jax import lax
from jax.experimental import pallas as pl
from jax.experimental.pallas import tpu as pltpu
from jax.experimental.pallas import tpu_sc as plsc

PAIR = 2
GP = 4
NSTREAM = 2
L = 16


def _make_gather(nw, nc, b, f, d):
    bpw = b // nw
    lp = PAIR * f
    pairs = bpw // PAIR
    groups = pairs // GP
    mesh = plsc.VectorSubcoreMesh(core_axis_name="c", subcore_axis_name="s")

    @functools.partial(
        pl.kernel,
        out_type=jax.ShapeDtypeStruct((b, f, d), jnp.float32),
        mesh=mesh,
        scratch_types=[
            pltpu.VMEM((bpw * f,), jnp.int32),
            pltpu.VMEM((2, GP, lp, d), jnp.float32),
            pltpu.SemaphoreType.DMA,
            pltpu.SemaphoreType.DMA,
        ],
        compiler_params=pltpu.CompilerParams(use_tc_tiling_on_sc=False),
    )
    def gather_kernel(idx_hbm, table_hbm, out_hbm, idx_v, rows_v, sem, semd):
        wid = lax.axis_index("s") * nc + lax.axis_index("c")
        b0 = wid * bpw
        pltpu.sync_copy(idx_hbm.at[wid], idx_v)

        def descs(g, slot):
            return [
                pltpu.make_async_copy(
                    table_hbm.at[idx_v.at[pl.ds((g * GP + j) * lp, lp)]],
                    rows_v.at[slot, j],
                    sem,
                )
                for j in range(NSTREAM)
            ]

        def fire(g, slot):
            for c in descs(g, slot):
                c.start()
            for j in range(NSTREAM, GP):
                base = (g * GP + j) * lp

                def win(w, carry, j=j):
                    vec = idx_v[pl.ds(base + w * L, L)]
                    for l in range(L):
                        pltpu.async_copy(
                            table_hbm.at[vec[l]],
                            rows_v.at[slot, j, w * L + l],
                            semd,
                        )
                    return carry

                lax.fori_loop(0, lp // L, win, 0)

        fire(0, 0)

        def body(g, carry):
            slot = lax.rem(g, 2)

            @pl.when(g + 1 < groups)
            def _():
                fire(g + 1, 1 - slot)

            for c in descs(g, slot):
                c.wait()
            for j in range(NSTREAM, GP):
                def wwait(w, carry, j=j):
                    for l in range(L):
                        pltpu.make_async_copy(
                            table_hbm.at[0],
                            rows_v.at[slot, j, w * L + l],
                            semd,
                        ).wait()
                    return carry

                lax.fori_loop(0, lp // L, wwait, 0)
            for j in range(GP):
                for p in range(PAIR):
                    pltpu.sync_copy(
                        rows_v.at[slot, j, pl.ds(p * f, f)],
                        out_hbm.at[b0 + (g * GP + j) * PAIR + p],
                    )
            return carry

        lax.fori_loop(0, groups, body, 0)

    return gather_kernel


def kernel(indices, weight):
    b, f = indices.shape
    v, d = weight.shape
    info = plsc.get_sparse_core_info()
    nc, ns = info.num_cores, info.num_subcores
    nw = nc * ns
    assert b % (nw * PAIR * GP) == 0
    idx2 = indices.reshape(nw, (b // nw) * f)
    out = _make_gather(nw, nc, b, f, d)(idx2, weight)
    return out


# speedup vs baseline: 1.0216x; 1.0216x over previous
import functools

import jax
import jax.numpy as jnp
from jax import lax
from jax.experimental import pallas as pl
from jax.experimental.pallas import tpu as pltpu
from jax.experimental.pallas import tpu_sc as plsc

PAIR = 2
GP = 4


def _make_gather(nw, nc, b, f, d):
    bpw = b // nw
    lp = PAIR * f
    pairs = bpw // PAIR
    groups = pairs // GP
    mesh = plsc.VectorSubcoreMesh(core_axis_name="c", subcore_axis_name="s")

    @functools.partial(
        pl.kernel,
        out_type=jax.ShapeDtypeStruct((b, f, d), jnp.float32),
        mesh=mesh,
        scratch_types=[
            pltpu.VMEM((bpw * f,), jnp.int32),
            pltpu.VMEM((2, GP, lp, d), jnp.float32),
            pltpu.SemaphoreType.DMA,
        ],
        compiler_params=pltpu.CompilerParams(use_tc_tiling_on_sc=False),
    )
    def gather_kernel(idx_hbm, table_hbm, out_hbm, idx_v, rows_v, sem):
        wid = lax.axis_index("s") * nc + lax.axis_index("c")
        b0 = wid * bpw
        pltpu.sync_copy(idx_hbm.at[wid], idx_v)

        def descs(g, slot):
            return [
                pltpu.make_async_copy(
                    table_hbm.at[idx_v.at[pl.ds((g * GP + j) * lp, lp)]],
                    rows_v.at[slot, j],
                    sem,
                )
                for j in range(GP)
            ]

        def fire(g, slot):
            for c in descs(g, slot):
                c.start()

        fire(0, 0)

        def body(g, carry):
            slot = lax.rem(g, 2)

            @pl.when(g + 1 < groups)
            def _():
                fire(g + 1, 1 - slot)

            for c in descs(g, slot):
                c.wait()
            for j in range(GP):
                for p in range(PAIR):
                    pltpu.sync_copy(
                        rows_v.at[slot, j, pl.ds(p * f, f)],
                        out_hbm.at[b0 + (g * GP + j) * PAIR + p],
                    )
            return carry

        lax.fori_loop(0, groups, body, 0)

    return gather_kernel


def kernel(indices, weight):
    b, f = indices.shape
    v, d = weight.shape
    info = plsc.get_sparse_core_info()
    nc, ns = info.num_cores, info.num_subcores
    nw = nc * ns
    assert b % (nw * PAIR * GP) == 0
    idx2 = indices.reshape(nw, (b // nw) * f)
    out = _make_gather(nw, nc, b, f, d)(idx2, weight)
    return out


# speedup vs baseline: 1.0232x; 1.0016x over previous
import functools

import jax
import jax.numpy as jnp
from jax import lax
from jax.experimental import pallas as pl
from jax.experimental.pallas import tpu as pltpu
from jax.experimental.pallas import tpu_sc as plsc

PAIR = 1
GP = 8


def _make_gather(nw, nc, b, f, d):
    bpw = b // nw
    lp = PAIR * f
    pairs = bpw // PAIR
    groups = pairs // GP
    mesh = plsc.VectorSubcoreMesh(core_axis_name="c", subcore_axis_name="s")

    @functools.partial(
        pl.kernel,
        out_type=jax.ShapeDtypeStruct((b, f, d), jnp.float32),
        mesh=mesh,
        scratch_types=[
            pltpu.VMEM((bpw, f), jnp.int32),
            pltpu.VMEM((2, GP, lp, d), jnp.float32),
            pltpu.SemaphoreType.DMA,
        ],
        compiler_params=pltpu.CompilerParams(use_tc_tiling_on_sc=False),
    )
    def gather_kernel(idx_hbm, table_hbm, out_hbm, idx_v, rows_v, sem):
        wid = lax.axis_index("s") * nc + lax.axis_index("c")
        b0 = wid * bpw
        pltpu.sync_copy(idx_hbm.at[pl.ds(b0, bpw)], idx_v)

        def descs(g, slot):
            return [
                pltpu.make_async_copy(
                    table_hbm.at[idx_v.at[g * GP + j]],
                    rows_v.at[slot, j],
                    sem,
                )
                for j in range(GP)
            ]

        def fire(g, slot):
            for c in descs(g, slot):
                c.start()

        fire(0, 0)

        def body(g, carry):
            slot = lax.rem(g, 2)

            @pl.when(g + 1 < groups)
            def _():
                fire(g + 1, 1 - slot)

            for c in descs(g, slot):
                c.wait()
            for j in range(GP):
                for p in range(PAIR):
                    pltpu.sync_copy(
                        rows_v.at[slot, j, pl.ds(p * f, f)],
                        out_hbm.at[b0 + (g * GP + j) * PAIR + p],
                    )
            return carry

        lax.fori_loop(0, groups, body, 0)

    return gather_kernel


def kernel(indices, weight):
    b, f = indices.shape
    v, d = weight.shape
    info = plsc.get_sparse_core_info()
    nc, ns = info.num_cores, info.num_subcores
    nw = nc * ns
    assert b % (nw * PAIR * GP) == 0
    out = _make_gather(nw, nc, b, f, d)(indices, weight)
    return out
